# baseline (speedup 1.0000x reference)
import functools

import jax
import jax.numpy as jnp
from jax import lax
from jax.experimental import pallas as pl
from jax.experimental.pallas import tpu as pltpu
from jax.experimental.pallas import tpu_sc as plsc

_BETA = 0.25
_MB = 1024

_NC, _NS, _LANES = 2, 16, 16
_NW = _NC * _NS
_CH = 128


def _argmin_body(z_ref, zbf_ref, zsq_ref, e_ref, esq_ref,
                 idx_ref, cout_ref, counts_ref, *, n_codes, n_rows):
    i = pl.program_id(0)
    nsteps = pl.num_programs(0)

    @pl.when(i == 0)
    def _init():
        counts_ref[...] = jnp.zeros_like(counts_ref)

    zb = z_ref[...]
    zbf = zbf_ref[...]
    e = e_ref[...]
    zsq = zsq_ref[...]
    esq = esq_ref[...]

    mm = jax.lax.dot_general(zbf, e, (((1,), (1,)), ((), ())),
                             preferred_element_type=jnp.float32)
    d = (zsq + esq) + mm

    half = n_codes // 2
    d0 = d[:, :half]
    d1 = d[:, half:]
    m0 = jnp.min(d0, axis=1, keepdims=True)
    m1 = jnp.min(d1, axis=1, keepdims=True)
    m0_bf = m0.astype(jnp.bfloat16).astype(jnp.float32)
    take = m1 < m0_bf

    d_w = jnp.where(take, d1, d0)
    m_w = jnp.where(take, m1, m0)
    jiota = jax.lax.broadcasted_iota(jnp.int32, (zb.shape[0], half), 1)
    i_rel = jnp.min(jnp.where(d_w == m_w, jiota, n_codes), axis=1)
    idx = (i_rel + jnp.where(take[:, 0], half, 0)).astype(jnp.int32)
    idx_ref[...] = idx

    onehot = (jiota == i_rel[:, None]).astype(jnp.float32)
    take_row = take.astype(jnp.float32).reshape(1, zb.shape[0])
    keep_row = 1.0 - take_row
    counts_ref[0:1, :half] += jax.lax.dot_general(
        keep_row, onehot, (((1,), (0,)), ((), ())),
        preferred_element_type=jnp.float32)
    counts_ref[0:1, half:] += jax.lax.dot_general(
        take_row, onehot, (((1,), (0,)), ((), ())),
        preferred_element_type=jnp.float32)

    @pl.when(i == nsteps - 1)
    def _finalize():
        cout_ref[...] = counts_ref[...]


def _sc_stage(idx, zf, table_pad):
    m_rows = zf.shape[0]
    bpw = m_rows // _NW
    n_chunks = bpw // _CH

    mesh = plsc.VectorSubcoreMesh(core_axis_name="c", subcore_axis_name="s",
                                  num_cores=_NC)

    @functools.partial(
        pl.kernel, mesh=mesh,
        out_type=jax.ShapeDtypeStruct((m_rows, 128), jnp.float32),
        scratch_types=[
            pltpu.VMEM((_CH,), jnp.int32),
            pltpu.VMEM((_CH, 128), jnp.float32),
            pltpu.SemaphoreType.DMA,
        ],
    )
    def k(idx_hbm, z_hbm, table_hbm, zq_hbm,
          idx_v, rows_v, sem):
        c = lax.axis_index("c")
        s = lax.axis_index("s")
        wid = s * _NC + c
        base = wid * bpw
        for kc in range(n_chunks):
            off = base + kc * _CH
            pltpu.sync_copy(idx_hbm.at[pl.ds(off, _CH)], idx_v)
            pltpu.async_copy(table_hbm.at[idx_v], rows_v, sem).wait()
            pltpu.sync_copy(rows_v, zq_hbm.at[pl.ds(off, _CH)])

    return k(idx, zf, table_pad)


def _fin_body(zq_ref, z_ref, ca_ref, cb_ref, loss_ref, perp_ref, *, n_total):
    diff = zq_ref[...] - z_ref[...]
    mean = jnp.sum(diff * diff) / n_total
    loss_ref[...] = jnp.full((1, 1), mean + _BETA * mean, jnp.float32)
    counts = ca_ref[...] + cb_ref[...]
    e_mean = counts * (1.0 / (n_total // 32))
    ent = -jnp.sum(e_mean * jnp.log(e_mean + 1e-10))
    perp_ref[...] = jnp.full((1, 1), jnp.exp(ent), jnp.float32)


def kernel(z, embedding_weight):
    e_dim = z.shape[-1]
    zf = z.reshape(-1, e_dim)
    m = zf.shape[0]
    n = embedding_weight.shape[0]
    n_total = m * e_dim

    zbf = (-2.0 * zf).astype(jnp.bfloat16)
    zsq = jnp.sum(zf ** 2, axis=1).reshape(m, 1)
    esq = jnp.sum(embedding_weight ** 2, axis=1).reshape(1, n)

    mh = m // 2

    def tc1(zf_h, zbf_h, zsq_h):
        return pl.pallas_call(
            functools.partial(_argmin_body, n_codes=n, n_rows=m),
            grid=(mh // _MB,),
            in_specs=[
                pl.BlockSpec((_MB, e_dim), lambda i: (i, 0)),
                pl.BlockSpec((_MB, e_dim), lambda i: (i, 0)),
                pl.BlockSpec((_MB, 1), lambda i: (i, 0)),
                pl.BlockSpec((n, e_dim), lambda i: (0, 0)),
                pl.BlockSpec((1, n), lambda i: (0, 0)),
            ],
            out_specs=[
                pl.BlockSpec((_MB,), lambda i: (i,)),
                pl.BlockSpec((1, n), lambda i: (0, 0)),
            ],
            out_shape=[
                jax.ShapeDtypeStruct((mh,), jnp.int32),
                jax.ShapeDtypeStruct((1, n), jnp.float32),
            ],
            scratch_shapes=[
                pltpu.VMEM((1, n), jnp.float32),
            ],
        )(zf_h, zbf_h, zsq_h, embedding_weight, esq)

    table_pad = jnp.pad(embedding_weight, ((0, 0), (0, 128 - e_dim)))
    idx_a, counts_a = tc1(zf[:mh], zbf[:mh], zsq[:mh])
    zq_a = _sc_stage(idx_a, zf[:mh], table_pad)
    idx_b, counts_b = tc1(zf[mh:], zbf[mh:], zsq[mh:])
    zq_b = _sc_stage(idx_b, zf[mh:], table_pad)
    idx = jnp.concatenate([idx_a, idx_b])
    zq_st = jnp.concatenate([zq_a[:, :e_dim], zq_b[:, :e_dim]])

    loss2d, perp2d = pl.pallas_call(
        functools.partial(_fin_body, n_total=n_total),
        out_shape=[
            jax.ShapeDtypeStruct((1, 1), jnp.float32),
            jax.ShapeDtypeStruct((1, 1), jnp.float32),
        ],
    )(zq_st, zf, counts_a, counts_b)

    loss = loss2d.reshape(())
    perplexity = perp2d.reshape(())
    return (loss, zq_st.reshape(z.shape), idx, perplexity)


# speedup vs baseline: 1.1000x; 1.1000x over previous
import functools

import jax
import jax.numpy as jnp
from jax import lax
from jax.experimental import pallas as pl
from jax.experimental.pallas import tpu as pltpu
from jax.experimental.pallas import tpu_sc as plsc

_BETA = 0.25
_MB = 1024

_NC, _NS, _LANES = 2, 16, 16
_NW = _NC * _NS
_CH = 128


def _argmin_body(z_ref, zbf_ref, zsq_ref, e_ref, esq_ref,
                 idx_ref, perp_ref, counts_ref, *, n_codes, n_rows):
    i = pl.program_id(0)
    nsteps = pl.num_programs(0)

    @pl.when(i == 0)
    def _init():
        counts_ref[...] = jnp.zeros_like(counts_ref)

    zb = z_ref[...]
    zbf = zbf_ref[...]
    e = e_ref[...]
    zsq = zsq_ref[...]
    esq = esq_ref[...]

    mm = jax.lax.dot_general(zbf, e, (((1,), (1,)), ((), ())),
                             preferred_element_type=jnp.float32)
    d = (zsq + esq) + mm

    half = n_codes // 2
    d0 = d[:, :half]
    d1 = d[:, half:]
    m0 = jnp.min(d0, axis=1, keepdims=True)
    m1 = jnp.min(d1, axis=1, keepdims=True)
    m0_bf = m0.astype(jnp.bfloat16).astype(jnp.float32)
    take = m1 < m0_bf

    d_w = jnp.where(take, d1, d0)
    m_w = jnp.where(take, m1, m0)
    jiota = jax.lax.broadcasted_iota(jnp.int32, (zb.shape[0], half), 1)
    i_rel = jnp.min(jnp.where(d_w == m_w, jiota, n_codes), axis=1)
    idx = (i_rel + jnp.where(take[:, 0], half, 0)).astype(jnp.int32)
    idx_ref[...] = idx

    onehot = (jiota == i_rel[:, None]).astype(jnp.float32)
    take_row = take.astype(jnp.float32).reshape(1, zb.shape[0])
    keep_row = 1.0 - take_row
    counts_ref[0:1, :half] += jax.lax.dot_general(
        keep_row, onehot, (((1,), (0,)), ((), ())),
        preferred_element_type=jnp.float32)
    counts_ref[0:1, half:] += jax.lax.dot_general(
        take_row, onehot, (((1,), (0,)), ((), ())),
        preferred_element_type=jnp.float32)

    @pl.when(i == nsteps - 1)
    def _finalize():
        e_mean = counts_ref[...] * (1.0 / n_rows)
        ent = -jnp.sum(e_mean * jnp.log(e_mean + 1e-10))
        perp_ref[...] = jnp.full((1, 1), jnp.exp(ent), jnp.float32)


def _sc_stage(idx, zf, table_pad):
    m_rows = zf.shape[0]
    bpw = m_rows // _NW
    n_chunks = bpw // _CH

    mesh = plsc.VectorSubcoreMesh(core_axis_name="c", subcore_axis_name="s",
                                  num_cores=_NC)

    @functools.partial(
        pl.kernel, mesh=mesh,
        out_type=jax.ShapeDtypeStruct((m_rows, 128), jnp.float32),
        scratch_types=[
            pltpu.VMEM((_CH,), jnp.int32),
            pltpu.VMEM((_CH, 128), jnp.float32),
            pltpu.SemaphoreType.DMA,
        ],
    )
    def k(idx_hbm, z_hbm, table_hbm, zq_hbm,
          idx_v, rows_v, sem):
        c = lax.axis_index("c")
        s = lax.axis_index("s")
        wid = s * _NC + c
        base = wid * bpw
        for kc in range(n_chunks):
            off = base + kc * _CH
            pltpu.sync_copy(idx_hbm.at[pl.ds(off, _CH)], idx_v)
            pltpu.async_copy(table_hbm.at[idx_v], rows_v, sem).wait()
            pltpu.sync_copy(rows_v, zq_hbm.at[pl.ds(off, _CH)])

    return k(idx, zf, table_pad)


def _loss_body(zq_ref, z_ref, loss_ref, *, n_total):
    diff = zq_ref[...] - z_ref[...]
    mean = jnp.sum(diff * diff) / n_total
    loss_ref[...] = jnp.full((1, 1), mean + _BETA * mean, jnp.float32)


def kernel(z, embedding_weight):
    e_dim = z.shape[-1]
    zf = z.reshape(-1, e_dim)
    m = zf.shape[0]
    n = embedding_weight.shape[0]
    n_total = m * e_dim

    zbf = (-2.0 * zf).astype(jnp.bfloat16)
    zsq = jnp.sum(zf ** 2, axis=1).reshape(m, 1)
    esq = jnp.sum(embedding_weight ** 2, axis=1).reshape(1, n)

    idx, perp2d = pl.pallas_call(
        functools.partial(_argmin_body, n_codes=n, n_rows=m),
        grid=(m // _MB,),
        in_specs=[
            pl.BlockSpec((_MB, e_dim), lambda i: (i, 0)),
            pl.BlockSpec((_MB, e_dim), lambda i: (i, 0)),
            pl.BlockSpec((_MB, 1), lambda i: (i, 0)),
            pl.BlockSpec((n, e_dim), lambda i: (0, 0)),
            pl.BlockSpec((1, n), lambda i: (0, 0)),
        ],
        out_specs=[
            pl.BlockSpec((_MB,), lambda i: (i,)),
            pl.BlockSpec((1, 1), lambda i: (0, 0)),
        ],
        out_shape=[
            jax.ShapeDtypeStruct((m,), jnp.int32),
            jax.ShapeDtypeStruct((1, 1), jnp.float32),
        ],
        scratch_shapes=[
            pltpu.VMEM((1, n), jnp.float32),
        ],
    )(zf, zbf, zsq, embedding_weight, esq)

    table_pad = jnp.pad(embedding_weight, ((0, 0), (0, 128 - e_dim)))
    zq_st = _sc_stage(idx, zf, table_pad)[:, :e_dim]

    loss2d = pl.pallas_call(
        functools.partial(_loss_body, n_total=n_total),
        out_shape=jax.ShapeDtypeStruct((1, 1), jnp.float32),
    )(zq_st, zf)

    loss = loss2d.reshape(())
    perplexity = perp2d.reshape(())
    return (loss, zq_st.reshape(z.shape), idx, perplexity)
